# baseline (speedup 1.0000x reference)
import functools

import jax
import jax.numpy as jnp
from jax import lax
from jax.experimental import pallas as pl
from jax.experimental.pallas import tpu as pltpu
from jax.experimental.pallas import tpu_sc as plsc

_TOP_K = 8


def _gate_topk_kernel(x_ref, w_ref, b_ref, idx_ref, m_ref):
    x = x_ref[...]
    w = w_ref[...]
    logits = jax.lax.dot_general(
        x, w, (((1,), (1,)), ((), ())), preferred_element_type=jnp.float32
    )
    scores = jax.nn.sigmoid(logits)
    biased = scores + b_ref[...]
    expert_ids = jax.lax.broadcasted_iota(jnp.int32, biased.shape, 1)
    neg_inf = jnp.float32(-jnp.inf)
    for k in range(_TOP_K):
        m = jnp.max(biased, axis=-1, keepdims=True)
        am = jnp.argmax(biased, axis=-1, keepdims=True)
        hot = expert_ids == am
        idx_ref[:, k : k + 1] = am.astype(jnp.int32)
        m_ref[:, k : k + 1] = m
        biased = jnp.where(hot, neg_inf, biased)


def _tc_gate_topk(x, gate_weight, bias2d):
    n_tokens, d_model = x.shape
    n_experts = gate_weight.shape[0]
    block_tokens = 2048
    grid = (n_tokens // block_tokens,)
    return pl.pallas_call(
        _gate_topk_kernel,
        grid=grid,
        in_specs=[
            pl.BlockSpec((block_tokens, d_model), lambda i: (i, 0)),
            pl.BlockSpec((n_experts, d_model), lambda i: (0, 0)),
            pl.BlockSpec((1, n_experts), lambda i: (0, 0)),
        ],
        out_specs=[
            pl.BlockSpec((block_tokens, _TOP_K), lambda i: (i, 0)),
            pl.BlockSpec((block_tokens, _TOP_K), lambda i: (i, 0)),
        ],
        out_shape=[
            jax.ShapeDtypeStruct((n_tokens, _TOP_K), jnp.int32),
            jax.ShapeDtypeStruct((n_tokens, _TOP_K), jnp.float32),
        ],
        compiler_params=pltpu.CompilerParams(
            dimension_semantics=("parallel",),
        ),
    )(x, gate_weight, bias2d)


def _sc_weights_kernel(n_total, idx_hbm, m_hbm, bias_hbm, out_hbm, biasv, idxv, mv, outv):
    info = plsc.get_sparse_core_info()
    nw = info.num_cores * info.num_subcores
    chunk = n_total // nw
    wid = lax.axis_index("s") * info.num_cores + lax.axis_index("c")
    base = wid * chunk
    pltpu.sync_copy(bias_hbm, biasv)
    pltpu.sync_copy(idx_hbm.at[pl.ds(base, chunk)], idxv)
    pltpu.sync_copy(m_hbm.at[pl.ds(base, chunk)], mv)
    lane = lax.iota(jnp.int32, 16)
    lo_half = lane < 8

    def body(i, _):
        iv = idxv[pl.ds(i * 16, 16)]
        mvec = mv[pl.ds(i * 16, 16)]
        bg = plsc.load_gather(biasv, [iv])
        w = mvec - bg
        tot_all = lax.reduce_sum_p.bind(w, axes=(0,))
        tot0 = lax.reduce_sum_p.bind(jnp.where(lo_half, w, 0.0), axes=(0,))
        tot = jnp.where(lo_half, tot0, tot_all - tot0)
        outv[pl.ds(i * 16, 16)] = w * (1.0 / (tot + 1e-20))
        return 0

    lax.fori_loop(0, chunk // 16, body, 0)
    pltpu.sync_copy(outv, out_hbm.at[pl.ds(base, chunk)])


def _sc_weights(idx_flat, m_flat, bias):
    n_total = idx_flat.shape[0]
    info = plsc.get_sparse_core_info()
    nw = info.num_cores * info.num_subcores
    chunk = n_total // nw
    mesh = plsc.VectorSubcoreMesh(core_axis_name="c", subcore_axis_name="s")
    fn = functools.partial(
        pl.kernel,
        mesh=mesh,
        out_type=jax.ShapeDtypeStruct((n_total,), jnp.float32),
        scratch_types=[
            pltpu.VMEM((bias.shape[0],), jnp.float32),
            pltpu.VMEM((chunk,), jnp.int32),
            pltpu.VMEM((chunk,), jnp.float32),
            pltpu.VMEM((chunk,), jnp.float32),
        ],
        compiler_params=pltpu.CompilerParams(needs_layout_passes=False),
    )(functools.partial(_sc_weights_kernel, n_total))
    return fn(idx_flat, m_flat, bias)


def kernel(x, gate_weight, bias):
    n_tokens = x.shape[0]
    n_experts = gate_weight.shape[0]
    bias2d = bias.reshape(1, n_experts)
    idx, m = _tc_gate_topk(x, gate_weight, bias2d)
    wgt_flat = _sc_weights(idx.reshape(-1), m.reshape(-1), bias)
    return idx, wgt_flat.reshape(n_tokens, _TOP_K)


# speedup vs baseline: 1.3529x; 1.3529x over previous
import jax
import jax.numpy as jnp
from jax.experimental import pallas as pl
from jax.experimental.pallas import tpu as pltpu

_TOP_K = 8


def _gate_kernel(x_ref, w_ref, b_ref, idx_ref, wgt_ref):
    x = x_ref[...]
    w = w_ref[...]
    logits = jax.lax.dot_general(
        x, w, (((1,), (1,)), ((), ())), preferred_element_type=jnp.float32
    )
    neg_inf = jnp.float32(-jnp.inf)
    bias_row = b_ref[...]
    n = logits.shape[0]
    C = 256
    for c in range(n // C):
        lo = c * C
        scores = jax.nn.sigmoid(logits[lo : lo + C, :])
        biased = scores + bias_row
        expert_ids = jax.lax.broadcasted_iota(jnp.int32, biased.shape, 1)
        cols_i = []
        cols_s = []
        for _ in range(_TOP_K):
            am = jnp.argmax(biased, axis=-1, keepdims=True)
            hot = expert_ids == am
            s_k = jnp.sum(jnp.where(hot, scores, 0.0), axis=-1, keepdims=True)
            cols_i.append(am.astype(jnp.int32))
            cols_s.append(s_k)
            biased = jnp.where(hot, neg_inf, biased)
        sel = jnp.concatenate(cols_s, axis=-1)
        inv = 1.0 / (jnp.sum(sel, axis=-1, keepdims=True) + 1e-20)
        idx_ref[lo : lo + C, :] = jnp.concatenate(cols_i, axis=-1)
        wgt_ref[lo : lo + C, :] = sel * inv


def kernel(x, gate_weight, bias):
    n_tokens, d_model = x.shape
    n_experts = gate_weight.shape[0]
    block_tokens = 2048
    grid = (n_tokens // block_tokens,)
    bias2d = bias.reshape(1, n_experts)
    idx, wgt = pl.pallas_call(
        _gate_kernel,
        grid=grid,
        in_specs=[
            pl.BlockSpec((block_tokens, d_model), lambda i: (i, 0)),
            pl.BlockSpec((n_experts, d_model), lambda i: (0, 0)),
            pl.BlockSpec((1, n_experts), lambda i: (0, 0)),
        ],
        out_specs=[
            pl.BlockSpec((block_tokens, _TOP_K), lambda i: (i, 0)),
            pl.BlockSpec((block_tokens, _TOP_K), lambda i: (i, 0)),
        ],
        out_shape=[
            jax.ShapeDtypeStruct((n_tokens, _TOP_K), jnp.int32),
            jax.ShapeDtypeStruct((n_tokens, _TOP_K), jnp.float32),
        ],
        compiler_params=pltpu.CompilerParams(
            dimension_semantics=("parallel",),
        ),
    )(x, gate_weight, bias2d)
    return idx, wgt


# speedup vs baseline: 2.4018x; 1.7753x over previous
import jax
import jax.numpy as jnp
from jax.experimental import pallas as pl
from jax.experimental.pallas import tpu as pltpu

_TOP_K = 8


def _gate_kernel(x_ref, w_ref, b_ref, idx_ref, wgt_ref):
    x = x_ref[...]
    w = w_ref[...]
    logits_t = jax.lax.dot_general(
        w, x, (((1,), (1,)), ((), ())), preferred_element_type=jnp.float32
    )
    scores_t = jax.nn.sigmoid(logits_t)
    biased_t = scores_t + b_ref[...]
    expert_ids = jax.lax.broadcasted_iota(jnp.int32, biased_t.shape, 0)
    neg_inf = jnp.float32(-jnp.inf)
    big_i = jnp.int32(64)
    cols_s = []
    for k in range(_TOP_K):
        m = jnp.max(biased_t, axis=0, keepdims=True)
        hot = biased_t == m
        am = jnp.min(jnp.where(hot, expert_ids, big_i), axis=0, keepdims=True)
        first = expert_ids == am
        s_k = jnp.sum(jnp.where(first, scores_t, 0.0), axis=0, keepdims=True)
        idx_ref[k : k + 1, :] = am
        cols_s.append(s_k)
        biased_t = jnp.where(first, neg_inf, biased_t)
    total = cols_s[0]
    for k in range(1, _TOP_K):
        total = total + cols_s[k]
    inv = 1.0 / (total + 1e-20)
    for k in range(_TOP_K):
        wgt_ref[k : k + 1, :] = cols_s[k] * inv


def kernel(x, gate_weight, bias):
    n_tokens, d_model = x.shape
    n_experts = gate_weight.shape[0]
    block_tokens = 2048
    grid = (n_tokens // block_tokens,)
    bias_col = bias.reshape(n_experts, 1)
    idx_t, wgt_t = pl.pallas_call(
        _gate_kernel,
        grid=grid,
        in_specs=[
            pl.BlockSpec((block_tokens, d_model), lambda i: (i, 0)),
            pl.BlockSpec((n_experts, d_model), lambda i: (0, 0)),
            pl.BlockSpec((n_experts, 1), lambda i: (0, 0)),
        ],
        out_specs=[
            pl.BlockSpec((_TOP_K, block_tokens), lambda i: (0, i)),
            pl.BlockSpec((_TOP_K, block_tokens), lambda i: (0, i)),
        ],
        out_shape=[
            jax.ShapeDtypeStruct((_TOP_K, n_tokens), jnp.int32),
            jax.ShapeDtypeStruct((_TOP_K, n_tokens), jnp.float32),
        ],
        compiler_params=pltpu.CompilerParams(
            dimension_semantics=("parallel",),
        ),
    )(x, gate_weight, bias_col)
    return idx_t.T, wgt_t.T
